# baseline (device time: 36000 ns/iter reference)
import jax
import jax.numpy as jnp
from jax import lax
from jax.experimental import pallas as pl
from jax.experimental.pallas import tpu as pltpu

N_X = 2
C = 8


def kernel(Q, K, V):
    b, s, h, d = Q.shape
    bh = b * h
    bh_c = bh // C
    s_full = N_X * s
    h_s = s // 2
    scale = d ** -0.5

    qt = (
        jnp.transpose(Q, (0, 2, 1, 3)).reshape(bh, s, d) * scale
    ).astype(jnp.bfloat16)
    kt = jnp.transpose(K, (0, 2, 3, 1)).reshape(bh, d, s).astype(jnp.bfloat16)
    vt = jnp.transpose(V, (0, 2, 3, 1)).reshape(bh, d, s).astype(jnp.bfloat16)

    def body(q_ref, kt_ref, vt_ref, o_ref, kvf, l_acc,
             sx_send, sx_recv, sy_send, sy_recv):
        my_x = lax.axis_index("x")
        my_y = lax.axis_index("y")
        nbr_x = 1 - my_x
        nbr_y = 1 - my_y

        barrier = pltpu.get_barrier_semaphore()
        pl.semaphore_signal(
            barrier, inc=1,
            device_id=(nbr_x, my_y), device_id_type=pl.DeviceIdType.MESH,
        )
        pl.semaphore_signal(
            barrier, inc=1,
            device_id=(my_x, nbr_y), device_id_type=pl.DeviceIdType.MESH,
        )
        pl.semaphore_wait(barrier, 2)

        off = my_x * s
        nbr_off = nbr_x * s
        p1 = off + my_y * h_s
        p2 = nbr_off + my_y * h_s

        kvf[0, :, :, pl.ds(off, s)] = kt_ref[...]
        kvf[1, :, :, pl.ds(off, s)] = vt_ref[...]

        p1_rdmas = []
        for c in range(C):
            r = pltpu.make_async_remote_copy(
                src_ref=kvf.at[:, pl.ds(c * bh_c, bh_c), :, pl.ds(p1, h_s)],
                dst_ref=kvf.at[:, pl.ds(c * bh_c, bh_c), :, pl.ds(p1, h_s)],
                send_sem=sx_send.at[c],
                recv_sem=sx_recv.at[c],
                device_id=(nbr_x, my_y),
                device_id_type=pl.DeviceIdType.MESH,
            )
            r.start()
            p1_rdmas.append(r)

        def partial_chunk(c, start):
            for i in range(c * bh_c, (c + 1) * bh_c):
                qm = q_ref[i]
                kmt = kvf[0, i, :, pl.ds(start, s)]
                vmt = kvf[1, i, :, pl.ds(start, s)]
                sc = lax.dot_general(
                    qm, kmt, (((1,), (0,)), ((), ())),
                    preferred_element_type=jnp.float32,
                )
                p = jnp.exp(sc)
                l = jnp.sum(p, axis=1, keepdims=True)
                o = lax.dot_general(
                    p.astype(jnp.bfloat16), vmt,
                    (((1,), (1,)), ((), ())),
                    preferred_element_type=jnp.float32,
                )
                yield i, o, l

        p2_rdmas = []
        for c in range(C):
            p1_rdmas[c].wait_recv()
            r = pltpu.make_async_remote_copy(
                src_ref=kvf.at[:, pl.ds(c * bh_c, bh_c), :, pl.ds(p2, h_s)],
                dst_ref=kvf.at[:, pl.ds(c * bh_c, bh_c), :, pl.ds(p2, h_s)],
                send_sem=sy_send.at[c],
                recv_sem=sy_recv.at[c],
                device_id=(my_x, nbr_y),
                device_id_type=pl.DeviceIdType.MESH,
            )
            r.start()
            p2_rdmas.append(r)
            for i, o, l in partial_chunk(c, off):
                o_ref[i] = o
                l_acc[i] = l

        for c in range(C):
            p2_rdmas[c].wait_recv()
            for i, o, l in partial_chunk(c, nbr_off):
                o_ref[i] = (o_ref[i] + o) / (l_acc[i] + l)

        for c in range(C):
            p1_rdmas[c].wait_send()
            p2_rdmas[c].wait_send()

    out3 = pl.pallas_call(
        body,
        out_shape=jax.ShapeDtypeStruct((bh, s, d), jnp.float32),
        in_specs=[pl.BlockSpec(memory_space=pltpu.VMEM)] * 3,
        out_specs=pl.BlockSpec(memory_space=pltpu.VMEM),
        scratch_shapes=[
            pltpu.VMEM((2, bh, d, s_full), jnp.bfloat16),
            pltpu.VMEM((bh, s, 1), jnp.float32),
            pltpu.SemaphoreType.DMA((C,)),
            pltpu.SemaphoreType.DMA((C,)),
            pltpu.SemaphoreType.DMA((C,)),
            pltpu.SemaphoreType.DMA((C,)),
        ],
        compiler_params=pltpu.CompilerParams(collective_id=0),
    )(qt, kt, vt)

    return jnp.transpose(out3.reshape(b, h, s, d), (0, 2, 1, 3))


# device time: 30070 ns/iter; 1.1972x vs baseline; 1.1972x over previous
import jax
import jax.numpy as jnp
from jax import lax
from jax.experimental import pallas as pl
from jax.experimental.pallas import tpu as pltpu

N_X = 2
C = 4


def kernel(Q, K, V):
    b, s, h, d = Q.shape
    bh = b * h
    bh_c = bh // C
    s_full = N_X * s
    scale2 = (d ** -0.5) * 1.4426950408889634

    qt = jnp.transpose(Q, (0, 2, 1, 3)).reshape(bh, s, d)
    kt = jnp.transpose(K, (0, 2, 3, 1)).reshape(bh, d, s)
    vt = jnp.transpose(V, (0, 2, 3, 1)).reshape(bh, d, s)
    h_s = s // 2

    def body(q_ref, kt_ref, vt_ref, o_ref, kvf,
             sx_send, sx_recv, sy_send, sy_recv):
        my_x = lax.axis_index("x")
        my_y = lax.axis_index("y")
        nbr_x = 1 - my_x
        nbr_y = 1 - my_y

        off = my_x * s
        nbr_off = nbr_x * s
        p1 = off + my_y * h_s
        p2 = nbr_off + my_y * h_s
        lo = my_y * h_s
        lo2 = nbr_y * h_s

        kvf[0, :, :, pl.ds(p1, h_s)] = kt_ref[:, :, pl.ds(lo, h_s)].astype(
            jnp.bfloat16)
        kvf[1, :, :, pl.ds(p1, h_s)] = vt_ref[:, :, pl.ds(lo, h_s)].astype(
            jnp.bfloat16)

        barrier = pltpu.get_barrier_semaphore()
        pl.semaphore_signal(
            barrier, inc=1,
            device_id=(nbr_x, my_y), device_id_type=pl.DeviceIdType.MESH,
        )
        pl.semaphore_signal(
            barrier, inc=1,
            device_id=(my_x, nbr_y), device_id_type=pl.DeviceIdType.MESH,
        )
        pl.semaphore_wait(barrier, 2)

        p1_rdmas = []
        for c in range(C):
            r = pltpu.make_async_remote_copy(
                src_ref=kvf.at[:, pl.ds(c * bh_c, bh_c), :, pl.ds(p1, h_s)],
                dst_ref=kvf.at[:, pl.ds(c * bh_c, bh_c), :, pl.ds(p1, h_s)],
                send_sem=sx_send.at[c],
                recv_sem=sx_recv.at[c],
                device_id=(nbr_x, my_y),
                device_id_type=pl.DeviceIdType.MESH,
            )
            r.start()
            p1_rdmas.append(r)

        kvf[0, :, :, pl.ds(off + lo2, h_s)] = kt_ref[:, :, pl.ds(lo2, h_s)].astype(
            jnp.bfloat16)
        kvf[1, :, :, pl.ds(off + lo2, h_s)] = vt_ref[:, :, pl.ds(lo2, h_s)].astype(
            jnp.bfloat16)

        def compute_chunk(c):
            for i in range(c * bh_c, (c + 1) * bh_c):
                qm = (q_ref[i] * scale2).astype(jnp.bfloat16)
                kmt = kvf[0, i]
                vmt = kvf[1, i]
                sc = lax.dot_general(
                    qm, kmt, (((1,), (0,)), ((), ())),
                    preferred_element_type=jnp.float32,
                )
                p = jnp.exp2(sc)
                l = jnp.sum(p, axis=1, keepdims=True)
                o = lax.dot_general(
                    p.astype(jnp.bfloat16), vmt,
                    (((1,), (1,)), ((), ())),
                    preferred_element_type=jnp.float32,
                )
                o_ref[i] = o / l

        p2_rdmas = []
        for c in range(C):
            p1_rdmas[c].wait_recv()
            r = pltpu.make_async_remote_copy(
                src_ref=kvf.at[:, pl.ds(c * bh_c, bh_c), :, pl.ds(p2, h_s)],
                dst_ref=kvf.at[:, pl.ds(c * bh_c, bh_c), :, pl.ds(p2, h_s)],
                send_sem=sy_send.at[c],
                recv_sem=sy_recv.at[c],
                device_id=(my_x, nbr_y),
                device_id_type=pl.DeviceIdType.MESH,
            )
            r.start()
            p2_rdmas.append(r)
            if c >= 1:
                p2_rdmas[c - 1].wait_recv()
                compute_chunk(c - 1)
        p2_rdmas[C - 1].wait_recv()
        compute_chunk(C - 1)

        for c in range(C):
            p1_rdmas[c].wait_send()
            p2_rdmas[c].wait_send()

    out3 = pl.pallas_call(
        body,
        out_shape=jax.ShapeDtypeStruct((bh, s, d), jnp.float32),
        in_specs=[pl.BlockSpec(memory_space=pltpu.VMEM)] * 3,
        out_specs=pl.BlockSpec(memory_space=pltpu.VMEM),
        scratch_shapes=[
            pltpu.VMEM((2, bh, d, s_full), jnp.bfloat16),
            pltpu.SemaphoreType.DMA((C,)),
            pltpu.SemaphoreType.DMA((C,)),
            pltpu.SemaphoreType.DMA((C,)),
            pltpu.SemaphoreType.DMA((C,)),
        ],
        compiler_params=pltpu.CompilerParams(collective_id=0),
    )(qt, kt, vt)

    return jnp.transpose(out3.reshape(b, h, s, d), (0, 2, 1, 3))
